# Initial kernel scaffold; baseline (speedup 1.0000x reference)
import jax
import jax.numpy as jnp
from jax.experimental import pallas as pl


def kernel(pred, y_true):
    raise NotImplementedError("write your pallas kernel here")


# speedup vs baseline: 2.5866x; 2.5866x over previous
import functools

import jax
import jax.numpy as jnp
from jax.experimental import pallas as pl
from jax.experimental.pallas import tpu as pltpu


def _nce_block(pred_ref, y_ref, out_ref):
    i = pl.program_id(0)
    j = pl.program_id(1)

    x = pred_ref[0]
    y = y_ref[0]
    c, bh, w = x.shape

    m = jnp.max(x, axis=0)
    e = jnp.exp(x - m[None])
    s = jnp.sum(e, axis=0)
    cls = jax.lax.broadcasted_iota(jnp.int32, (c, bh, w), 0)
    sel = jnp.sum(jnp.where(cls == y[None].astype(jnp.int32), e, 0.0), axis=0)
    partial = jnp.sum(sel / s)

    @pl.when(jnp.logical_and(i == 0, j == 0))
    def _():
        out_ref[0, 0] = 0.0

    out_ref[0, 0] += partial


@functools.partial(jax.jit, static_argnames=("block_h",))
def _nce_sum(pred, y_true, block_h=128):
    b, c, h, w = pred.shape
    grid = (b, h // block_h)
    out = pl.pallas_call(
        _nce_block,
        grid=grid,
        in_specs=[
            pl.BlockSpec((1, c, block_h, w), lambda i, j: (i, 0, j, 0)),
            pl.BlockSpec((1, block_h, w), lambda i, j: (i, j, 0)),
        ],
        out_specs=pl.BlockSpec(
            (1, 1), lambda i, j: (0, 0), memory_space=pltpu.SMEM
        ),
        out_shape=jax.ShapeDtypeStruct((1, 1), jnp.float32),
    )(pred, y_true)
    return out[0, 0]


def kernel(pred, y_true):
    b, c, h, w = pred.shape
    total = _nce_sum(pred, y_true.astype(jnp.int32))
    return total / jnp.float32(b * h * w)
